# Initial kernel scaffold; baseline (speedup 1.0000x reference)
import jax
import jax.numpy as jnp
from jax.experimental import pallas as pl


def kernel(predictions, targets):
    raise NotImplementedError("write your pallas kernel here")


# speedup vs baseline: 44390.8602x; 44390.8602x over previous
import functools

import jax
import jax.numpy as jnp
from jax import lax
from jax.experimental import pallas as pl
from jax.experimental.pallas import tpu as pltpu
from jax.experimental.pallas import tpu_sc as plsc

_N = 4096
_L = 16
_NCHUNKS = _N // _L
_NPAIRS = _N * (_N - 1) // 2 + _N * _N


def _loss_body(p_hbm, t_hbm, out_hbm, p_v, t_v, out_v):
    c = lax.axis_index("c")
    s = lax.axis_index("s")

    @pl.when(jnp.logical_and(c == 0, s == 0))
    def _():
        pltpu.sync_copy(p_hbm, p_v)
        pltpu.sync_copy(t_hbm, t_v)

        def step(i, carry):
            s1, s2 = carry
            d = p_v[pl.ds(i * _L, _L)] - t_v[pl.ds(i * _L, _L)]
            return s1 + d, s2 + d * d

        zeros = jnp.zeros((_L,), jnp.float32)
        s1, s2 = lax.fori_loop(0, _NCHUNKS, step, (zeros, zeros))
        lane = lax.iota(jnp.int32, _L)
        for sh in (8, 4, 2, 1):
            perm = jnp.bitwise_xor(lane, sh)
            s1 = s1 + s1[perm]
            s2 = s2 + s2[perm]
        out_v[...] = (2.0 * _N * s2 - s1 * s1) * (1.0 / _NPAIRS)
        pltpu.sync_copy(out_v, out_hbm)


@jax.jit
def _moco_loss(p_flat, t_flat):
    mesh = plsc.VectorSubcoreMesh(core_axis_name="c", subcore_axis_name="s")
    out = pl.kernel(
        _loss_body,
        out_type=jax.ShapeDtypeStruct((_L,), jnp.float32),
        mesh=mesh,
        scratch_types=[
            pltpu.VMEM((_N,), jnp.float32),
            pltpu.VMEM((_N,), jnp.float32),
            pltpu.VMEM((_L,), jnp.float32),
        ],
    )(p_flat, t_flat)
    return out[0]


def kernel(predictions, targets):
    return _moco_loss(predictions.reshape(_N), targets.reshape(_N))


# speedup vs baseline: 50922.8964x; 1.1471x over previous
import functools

import jax
import jax.numpy as jnp
from jax import lax
from jax.experimental import pallas as pl
from jax.experimental.pallas import tpu as pltpu
from jax.experimental.pallas import tpu_sc as plsc

_N = 4096
_L = 16
_NCHUNKS = _N // _L
_NPAIRS = _N * (_N - 1) // 2 + _N * _N


_UNROLL = 4


def _loss_body(p_hbm, t_hbm, out_hbm, p_v, t_v, out_v, sem):
    c = lax.axis_index("c")
    s = lax.axis_index("s")

    @pl.when(jnp.logical_and(c == 0, s == 0))
    def _():
        cp_p = pltpu.async_copy(p_hbm, p_v, sem)
        cp_t = pltpu.async_copy(t_hbm, t_v, sem)
        cp_p.wait()
        cp_t.wait()

        def step(i, carry):
            acc = list(carry)
            base = i * (_UNROLL * _L)
            for u in range(_UNROLL):
                sl = pl.ds(base + u * _L, _L)
                d = p_v[sl] - t_v[sl]
                acc[2 * u] = acc[2 * u] + d
                acc[2 * u + 1] = acc[2 * u + 1] + d * d
            return tuple(acc)

        zeros = jnp.zeros((_L,), jnp.float32)
        acc = lax.fori_loop(
            0, _NCHUNKS // _UNROLL, step, (zeros,) * (2 * _UNROLL)
        )
        s1 = acc[0] + acc[2] + acc[4] + acc[6]
        s2 = acc[1] + acc[3] + acc[5] + acc[7]
        lane = lax.iota(jnp.int32, _L)
        for sh in (8, 4, 2, 1):
            perm = jnp.bitwise_xor(lane, sh)
            s1 = s1 + s1[perm]
            s2 = s2 + s2[perm]
        out_v[...] = (2.0 * _N * s2 - s1 * s1) * (1.0 / _NPAIRS)
        pltpu.sync_copy(out_v, out_hbm)


@jax.jit
def _moco_loss(p_flat, t_flat):
    mesh = plsc.VectorSubcoreMesh(
        core_axis_name="c", subcore_axis_name="s", num_cores=1
    )
    out = pl.kernel(
        _loss_body,
        out_type=jax.ShapeDtypeStruct((_L,), jnp.float32),
        mesh=mesh,
        scratch_types=[
            pltpu.VMEM((_N,), jnp.float32),
            pltpu.VMEM((_N,), jnp.float32),
            pltpu.VMEM((_L,), jnp.float32),
            pltpu.SemaphoreType.DMA,
        ],
    )(p_flat, t_flat)
    return out[0]


def kernel(predictions, targets):
    return _moco_loss(predictions.reshape(_N), targets.reshape(_N))
